# Initial kernel scaffold; baseline (speedup 1.0000x reference)
import jax
import jax.numpy as jnp
from jax.experimental import pallas as pl


def kernel(s, v, neighbor_idx, rel_unit, distances, rbf, W1, b1, W2, b2, Wvs, Wsv, Wu1, bu1, Wu2, bu2, Wvm, gamma_s, beta_s, gamma_v, beta_v):
    raise NotImplementedError("write your pallas kernel here")


# speedup vs baseline: 12.6232x; 12.6232x over previous
import functools

import jax
import jax.numpy as jnp
from jax import lax
from jax.experimental import pallas as pl
from jax.experimental.pallas import tpu as pltpu
from jax.experimental.pallas import tpu_sc as plsc

CUTOFF = 0.8
EPS_V = 1e-6
LN_EPS = 1e-5

_NC = 2
_NS = 16
_NW = _NC * _NS

_CHUNK = 128


def _sc_gather_body(nchunk, iters, tab_hbm, idx_hbm, ej_hbm,
                    idx_v, rows, sem):
    wid = lax.axis_index("s") * _NC + lax.axis_index("c")

    def step(t, carry):
        ci = wid + _NW * t

        @pl.when(ci < nchunk)
        def _():
            off = ci * _CHUNK
            pltpu.sync_copy(idx_hbm.at[pl.ds(off, _CHUNK)], idx_v)
            pltpu.async_copy(tab_hbm.at[idx_v], rows, sem).wait()
            pltpu.sync_copy(rows, ej_hbm.at[pl.ds(off, _CHUNK)])

        return carry

    lax.fori_loop(0, iters, step, 0)


def _sc_gather(table, idxg):
    e = idxg.shape[0]
    cols = table.shape[1]
    nchunk = e // _CHUNK
    iters = -(-nchunk // _NW)
    body = functools.partial(_sc_gather_body, nchunk, iters)
    fn = pl.kernel(
        body,
        out_type=jax.ShapeDtypeStruct((e, cols), jnp.float32),
        mesh=plsc.VectorSubcoreMesh(core_axis_name="c", subcore_axis_name="s"),
        scratch_types=[
            pltpu.VMEM((_CHUNK,), jnp.int32),
            pltpu.VMEM((_CHUNK, cols), jnp.float32),
            pltpu.SemaphoreType.DMA,
        ],
    )
    return fn(table, idxg)


def _tc_body(n_t, n_k, s_dim,
             si_ref, vi_ref, ej_ref, rbf_ref, ru_ref, d_ref,
             a1i_ref, a1j_ref, a1r_ref, b1_ref,
             a2ss_ref, a2sv_ref, a2vv_ref, a2vs_ref,
             b2ss_ref, b2sv_ref, b2vv_ref, b2vs_ref,
             avs_ref, asv_ref, au1_ref, bu1_ref, au2_ref, bu2_ref,
             avm_ref, gs_ref, bs_ref, gv_ref, bv_ref,
             so_ref, vx_ref, vy_ref, vz_ref):
    f32 = jnp.float32
    si = si_ref[...]
    ej = ej_ref[...]
    sj = ej[:, :s_dim]
    vj = ej[:, s_dim:]
    rbf = rbf_ref[...]
    ru = ru_ref[...]
    dist = d_ref[...]
    v_dim = avm_ref.shape[0]

    t_i = jnp.dot(si, a1i_ref[...], preferred_element_type=f32)
    t_i = jnp.broadcast_to(t_i[:, None, :], (n_t, n_k, t_i.shape[1]))
    t_i = t_i.reshape(n_t * n_k, -1)
    pre = (t_i
           + jnp.dot(sj, a1j_ref[...], preferred_element_type=f32)
           + jnp.dot(rbf, a1r_ref[...], preferred_element_type=f32)
           + b1_ref[...])
    h = pre * jax.nn.sigmoid(pre)

    g_ss = jnp.dot(h, a2ss_ref[...], preferred_element_type=f32) + b2ss_ref[...]
    g_sv = jnp.dot(h, a2sv_ref[...], preferred_element_type=f32) + b2sv_ref[...]
    g_vv = jnp.dot(h, a2vv_ref[...], preferred_element_type=f32) + b2vv_ref[...]
    g_vs = jnp.dot(h, a2vs_ref[...], preferred_element_type=f32) + b2vs_ref[...]

    rux = ru[:, 0:1]
    ruy = ru[:, 1:2]
    ruz = ru[:, 2:3]
    vjx = vj[:, 0 * v_dim:1 * v_dim]
    vjy = vj[:, 1 * v_dim:2 * v_dim]
    vjz = vj[:, 2 * v_dim:3 * v_dim]

    vproj = vjx * rux + vjy * ruy + vjz * ruz
    vps = jnp.dot(vproj, avs_ref[...], preferred_element_type=f32)

    mask = (dist < CUTOFF).astype(f32)
    scalar_edge = (g_ss * sj + g_sv * vps) * mask

    stv = jnp.dot(sj, asv_ref[...], preferred_element_type=f32)
    vex = (g_vv * vjx + g_vs * (stv * rux)) * mask
    vey = (g_vv * vjy + g_vs * (stv * ruy)) * mask
    vez = (g_vv * vjz + g_vs * (stv * ruz)) * mask

    mcnt = mask.reshape(n_t, n_k, 1).sum(axis=1)
    denom = jnp.maximum(mcnt, 1.0)
    sm = scalar_edge.reshape(n_t, n_k, -1).sum(axis=1) / denom
    vmx = vex.reshape(n_t, n_k, -1).sum(axis=1) / denom
    vmy = vey.reshape(n_t, n_k, -1).sum(axis=1) / denom
    vmz = vez.reshape(n_t, n_k, -1).sum(axis=1) / denom

    u = jnp.dot(sm, au1_ref[...], preferred_element_type=f32) + bu1_ref[...]
    u = u * jax.nn.sigmoid(u)
    su = jnp.dot(u, au2_ref[...], preferred_element_type=f32) + bu2_ref[...]
    sres = si + su
    mu = sres.mean(axis=-1, keepdims=True)
    var = ((sres - mu) ** 2).mean(axis=-1, keepdims=True)
    so_ref[...] = (sres - mu) * jax.lax.rsqrt(var + LN_EPS) * gs_ref[...] + bs_ref[...]

    avm = avm_ref[...]
    xx = vi_ref[:, 0 * v_dim:1 * v_dim] + jnp.dot(vmx, avm, preferred_element_type=f32)
    xy = vi_ref[:, 1 * v_dim:2 * v_dim] + jnp.dot(vmy, avm, preferred_element_type=f32)
    xz = vi_ref[:, 2 * v_dim:3 * v_dim] + jnp.dot(vmz, avm, preferred_element_type=f32)
    mag = jnp.maximum(jnp.sqrt(xx * xx + xy * xy + xz * xz), EPS_V)
    mmu = mag.mean(axis=-1, keepdims=True)
    mvar = ((mag - mmu) ** 2).mean(axis=-1, keepdims=True)
    magn = (mag - mmu) * jax.lax.rsqrt(mvar + LN_EPS) * gv_ref[...] + bv_ref[...]
    r = magn / mag
    vx_ref[...] = xx * r
    vy_ref[...] = xy * r
    vz_ref[...] = xz * r


def _tc_call_params(bn, n_k, s_dim, v_dim, r_dim, n_t):
    grid = (bn // n_t,)
    tk = n_t * n_k

    def node(c):
        return pl.BlockSpec((n_t, c), lambda i: (i, 0))

    def edge(c):
        return pl.BlockSpec((tk, c), lambda i: (i, 0))

    def w(r, c):
        return pl.BlockSpec((r, c), lambda i: (0, 0))

    h_dim = s_dim
    in_specs = [
        node(s_dim),
        node(3 * v_dim),
        edge(128),
        edge(r_dim),
        edge(3),
        edge(1),
        w(s_dim, h_dim),
        w(s_dim, h_dim),
        w(r_dim, h_dim),
        w(1, h_dim),
        w(h_dim, s_dim),
        w(h_dim, s_dim),
        w(h_dim, v_dim),
        w(h_dim, v_dim),
        w(1, s_dim),
        w(1, s_dim),
        w(1, v_dim),
        w(1, v_dim),
        w(v_dim, s_dim),
        w(s_dim, v_dim),
        w(s_dim, h_dim),
        w(1, h_dim),
        w(h_dim, s_dim),
        w(1, s_dim),
        w(v_dim, v_dim),
        w(1, s_dim),
        w(1, s_dim),
        w(1, v_dim),
        w(1, v_dim),
    ]
    out_specs = [
        node(s_dim),
        node(v_dim),
        node(v_dim),
        node(v_dim),
    ]
    out_shape = [
        jax.ShapeDtypeStruct((bn, s_dim), jnp.float32),
        jax.ShapeDtypeStruct((bn, v_dim), jnp.float32),
        jax.ShapeDtypeStruct((bn, v_dim), jnp.float32),
        jax.ShapeDtypeStruct((bn, v_dim), jnp.float32),
    ]
    return grid, in_specs, out_specs, out_shape


def kernel(s, v, neighbor_idx, rel_unit, distances, rbf,
           W1, b1, W2, b2, Wvs, Wsv, Wu1, bu1, Wu2, bu2, Wvm,
           gamma_s, beta_s, gamma_v, beta_v):
    b_dim, n_dim, s_dim = s.shape
    v_dim = v.shape[2]
    n_k = neighbor_idx.shape[2]
    r_dim = rbf.shape[3]
    bn = b_dim * n_dim
    e = bn * n_k

    s2 = s.reshape(bn, s_dim)
    v2 = jnp.swapaxes(v, 2, 3).reshape(bn, 3 * v_dim)
    pad = 128 - s_dim - 3 * v_dim
    table = jnp.concatenate(
        [s2, v2, jnp.zeros((bn, pad), jnp.float32)], axis=1)
    idxg = (neighbor_idx.astype(jnp.int32)
            + (jnp.arange(b_dim, dtype=jnp.int32) * n_dim)[:, None, None])
    idxg = idxg.reshape(e)

    ej = _sc_gather(table, idxg)

    rbf2 = rbf.reshape(e, r_dim)
    ru2 = rel_unit.reshape(e, 3)
    dist2 = distances.reshape(e, 1)

    n_t = 200
    grid, in_specs, out_specs, out_shape = _tc_call_params(
        bn, n_k, s_dim, v_dim, r_dim, n_t)
    body = functools.partial(_tc_body, n_t, n_k, s_dim)
    so, vx, vy, vz = pl.pallas_call(
        body,
        grid=grid,
        in_specs=in_specs,
        out_specs=out_specs,
        out_shape=out_shape,
    )(
        s2, v2, ej, rbf2, ru2, dist2,
        W1[:, :s_dim].T, W1[:, s_dim:2 * s_dim].T, W1[:, 2 * s_dim:].T,
        b1[None, :],
        W2[:s_dim].T, W2[s_dim:2 * s_dim].T,
        W2[2 * s_dim:2 * s_dim + v_dim].T, W2[2 * s_dim + v_dim:].T,
        b2[None, :s_dim], b2[None, s_dim:2 * s_dim],
        b2[None, 2 * s_dim:2 * s_dim + v_dim], b2[None, 2 * s_dim + v_dim:],
        Wvs.T, Wsv.T, Wu1.T, bu1[None, :], Wu2.T, bu2[None, :],
        Wvm.T,
        gamma_s[None, :], beta_s[None, :], gamma_v[None, :], beta_v[None, :],
    )

    s_out = so.reshape(b_dim, n_dim, s_dim)
    v_out = jnp.stack([vx, vy, vz], axis=-1).reshape(b_dim, n_dim, v_dim, 3)
    return (s_out, v_out)
